# baseline (speedup 1.0000x reference)
import functools

import jax
import jax.numpy as jnp
from jax import lax
from jax.experimental import pallas as pl
from jax.experimental.pallas import tpu as pltpu
from jax.experimental.pallas import tpu_sc as plsc

BATCH = 16384
N_FIELDS = 26
DIM = 32
TOTAL = BATCH * N_FIELDS
BT = 128
UNITS = TOTAL // BT
NC = 2
NS = 16
NW = NC * NS
UPW = UNITS // NW
NBUF = 4
AHEAD = 2
LANES = 16
BTILES = BATCH // BT
DG = DIM // 8


def _body(idx_hbm, table_hbm, out_hbm, idx_v, g_v, t_v, sem_g, sem_s):
    wid = lax.axis_index("s") * NC + lax.axis_index("c")
    u0 = wid * UPW

    pltpu.sync_copy(idx_hbm.at[pl.ds(u0, UPW)], idx_v)

    lane = lax.iota(jnp.int32, LANES)
    row_ids = [lane + j * LANES for j in range(BT // LANES)]

    def fire_gather(ul, buf):
        pltpu.async_copy(table_hbm.at[idx_v.at[ul]], g_v.at[buf], sem_g)

    def drain_gather(buf):
        pltpu.make_async_copy(
            table_hbm.at[idx_v.at[0]], g_v.at[buf], sem_g
        ).wait()

    def wait_stores(buf):
        for a in range(DG):
            pltpu.make_async_copy(
                out_hbm.at[0, a, 0], t_v.at[buf, a], sem_s.at[buf]
            ).wait()

    def transpose(buf):
        g = g_v.at[buf]
        for j in range(BT // LANES):
            for dh in range(2):
                vs = [
                    plsc.load_gather(
                        g,
                        [row_ids[j], jnp.full((LANES,), dh * 16 + k, jnp.int32)],
                    )
                    for k in range(16)
                ]
                for k in range(16):
                    d = dh * 16 + k
                    t_v[buf, d // 8, d % 8, pl.ds(j * LANES, LANES)] = vs[k]

    def fire_stores(u, buf):
        f = u // BTILES
        bt = lax.rem(u, BTILES)
        for a in range(DG):
            pltpu.async_copy(
                t_v.at[buf, a], out_hbm.at[f, a, bt], sem_s.at[buf]
            )

    for k in range(AHEAD):
        fire_gather(k, k)

    def step(i, _):
        for b in range(NBUF):
            ul = i * NBUF + b
            buf = b

            @pl.when(ul + AHEAD < UPW)
            def _():
                fire_gather(ul + AHEAD, (b + AHEAD) % NBUF)

            drain_gather(buf)

            @pl.when(ul >= NBUF)
            def _():
                wait_stores(buf)

            transpose(buf)
            fire_stores(u0 + ul, buf)
        return 0

    lax.fori_loop(0, UPW // NBUF, step, 0)

    for b in range(NBUF):
        wait_stores(b)


_emb = functools.partial(
    pl.kernel,
    out_type=jax.ShapeDtypeStruct((N_FIELDS, DG, BTILES, 8, BT), jnp.float32),
    mesh=plsc.VectorSubcoreMesh(core_axis_name="c", subcore_axis_name="s"),
    scratch_types=[
        pltpu.VMEM((UPW, BT), jnp.int32),
        pltpu.VMEM((NBUF, BT, DIM), jnp.float32),
        pltpu.VMEM((NBUF, DG, 8, BT), jnp.float32),
        pltpu.SemaphoreType.DMA,
        pltpu.SemaphoreType.DMA((NBUF,)),
    ],
    compiler_params=pltpu.CompilerParams(
        use_tc_tiling_on_sc=False, needs_layout_passes=False
    ),
)(_body)


def kernel(indices, embedding_table):
    idx2d = indices.astype(jnp.int32).T.reshape(UNITS, BT)
    out = _emb(idx2d, embedding_table)
    return out.transpose(2, 4, 0, 1, 3).reshape(BATCH, N_FIELDS, DIM)


# speedup vs baseline: 1.0322x; 1.0322x over previous
import functools

import jax
import jax.numpy as jnp
from jax import lax
from jax.experimental import pallas as pl
from jax.experimental.pallas import tpu as pltpu
from jax.experimental.pallas import tpu_sc as plsc

BATCH = 16384
N_FIELDS = 26
DIM = 32
TOTAL = BATCH * N_FIELDS
BT = 128
UNITS = TOTAL // BT
NC = 2
NS = 16
NW = NC * NS
UPW = UNITS // NW
NBUF = 2
AHEAD = 1
GPITCH = DIM + 1
LANES = 16
BTILES = BATCH // BT
DG = DIM // 8


def _body(idx_hbm, table_hbm, out_hbm, idx_v, g_v, t_v, sem_g, sem_s):
    wid = lax.axis_index("s") * NC + lax.axis_index("c")
    u0 = wid * UPW

    pltpu.sync_copy(idx_hbm.at[pl.ds(u0, UPW)], idx_v)

    lane = lax.iota(jnp.int32, LANES)
    row_ids = [lane + j * LANES for j in range(BT // LANES)]

    def fire_gather(ul, buf):
        pltpu.async_copy(table_hbm.at[idx_v.at[ul]], g_v.at[buf], sem_g)

    def drain_gather(buf):
        pltpu.make_async_copy(
            table_hbm.at[idx_v.at[0]], g_v.at[buf], sem_g
        ).wait()

    def wait_stores(buf):
        for a in range(DG):
            pltpu.make_async_copy(
                out_hbm.at[0, a, 0], t_v.at[buf, a], sem_s.at[buf]
            ).wait()

    def transpose(buf):
        g = g_v.at[buf]
        for j in range(BT // LANES):
            for dh in range(2):
                vs = [
                    plsc.load_gather(
                        g,
                        [row_ids[j], jnp.full((LANES,), dh * 16 + k, jnp.int32)],
                    )
                    for k in range(16)
                ]
                for k in range(16):
                    d = dh * 16 + k
                    t_v[buf, d // 8, d % 8, pl.ds(j * LANES, LANES)] = vs[k]

    def fire_stores(u, buf):
        f = u // BTILES
        bt = lax.rem(u, BTILES)
        for a in range(DG):
            pltpu.async_copy(
                t_v.at[buf, a], out_hbm.at[f, a, bt], sem_s.at[buf]
            )

    for k in range(AHEAD):
        fire_gather(k, k)

    def step(i, _):
        for b in range(NBUF):
            ul = i * NBUF + b
            buf = b

            @pl.when(ul + AHEAD < UPW)
            def _():
                fire_gather(ul + AHEAD, (b + AHEAD) % NBUF)

            drain_gather(buf)

            @pl.when(ul >= NBUF)
            def _():
                wait_stores(buf)

            transpose(buf)
            fire_stores(u0 + ul, buf)
        return 0

    lax.fori_loop(0, UPW // NBUF, step, 0)

    for b in range(NBUF):
        wait_stores(b)


_emb = functools.partial(
    pl.kernel,
    out_type=jax.ShapeDtypeStruct((N_FIELDS, DG, BTILES, 8, BT), jnp.float32),
    mesh=plsc.VectorSubcoreMesh(core_axis_name="c", subcore_axis_name="s"),
    scratch_types=[
        pltpu.VMEM((UPW, BT), jnp.int32),
        pltpu.VMEM((NBUF, BT, DIM), jnp.float32),
        pltpu.VMEM((NBUF, DG, 8, BT), jnp.float32),
        pltpu.SemaphoreType.DMA,
        pltpu.SemaphoreType.DMA((NBUF,)),
    ],
    compiler_params=pltpu.CompilerParams(
        use_tc_tiling_on_sc=False,
        needs_layout_passes=False,
        skip_device_barrier=True,
    ),
)(_body)


def kernel(indices, embedding_table):
    idx2d = indices.astype(jnp.int32).T.reshape(UNITS, BT)
    out = _emb(idx2d, embedding_table)
    return out.transpose(2, 4, 0, 1, 3).reshape(BATCH, N_FIELDS, DIM)


# speedup vs baseline: 1.1411x; 1.1055x over previous
import functools

import jax
import jax.numpy as jnp
from jax import lax
from jax.experimental import pallas as pl
from jax.experimental.pallas import tpu as pltpu
from jax.experimental.pallas import tpu_sc as plsc

BATCH = 16384
N_FIELDS = 26
DIM = 32
TOTAL = BATCH * N_FIELDS
BT = 128
UNITS = TOTAL // BT
NC = 2
NS = 16
NW = NC * NS
UPW = UNITS // NW
NBUF = 2
AHEAD = 1
GPITCH = DIM + 1
LANES = 16
BTILES = BATCH // BT
DG = DIM // 8


def _body(idx_hbm, table_hbm, out_hbm, idx_v, g_v, g2_v, t_v, sem_g, sem_s):
    wid = lax.axis_index("s") * NC + lax.axis_index("c")
    u0 = wid * UPW

    pltpu.sync_copy(idx_hbm.at[pl.ds(u0, UPW)], idx_v)

    lane = lax.iota(jnp.int32, LANES)
    row_ids = [lane + j * LANES for j in range(BT // LANES)]

    def fire_gather(ul, buf):
        pltpu.async_copy(table_hbm.at[idx_v.at[ul]], g_v.at[buf], sem_g)

    def drain_gather(buf):
        pltpu.make_async_copy(
            table_hbm.at[idx_v.at[0]], g_v.at[buf], sem_g
        ).wait()

    def wait_stores(buf):
        for a in range(DG):
            pltpu.make_async_copy(
                out_hbm.at[0, a, 0], t_v.at[buf, a], sem_s.at[buf]
            ).wait()

    def transpose(buf):
        for r in range(BT):
            for h in range(2):
                g2_v[buf, r, pl.ds(h * LANES, LANES)] = g_v[
                    buf, r, pl.ds(h * LANES, LANES)
                ]
        g2 = g2_v.at[buf]
        for j in range(BT // LANES):
            for dh in range(2):
                vs = [
                    plsc.load_gather(
                        g2,
                        [row_ids[j], jnp.full((LANES,), dh * 16 + k, jnp.int32)],
                    )
                    for k in range(16)
                ]
                for k in range(16):
                    d = dh * 16 + k
                    t_v[buf, d // 8, d % 8, pl.ds(j * LANES, LANES)] = vs[k]

    def fire_stores(u, buf):
        f = u // BTILES
        bt = lax.rem(u, BTILES)
        for a in range(DG):
            pltpu.async_copy(
                t_v.at[buf, a], out_hbm.at[f, a, bt], sem_s.at[buf]
            )

    for k in range(AHEAD):
        fire_gather(k, k)

    def step(i, _):
        for b in range(NBUF):
            ul = i * NBUF + b
            buf = b

            @pl.when(ul + AHEAD < UPW)
            def _():
                fire_gather(ul + AHEAD, (b + AHEAD) % NBUF)

            drain_gather(buf)

            @pl.when(ul >= NBUF)
            def _():
                wait_stores(buf)

            transpose(buf)
            fire_stores(u0 + ul, buf)
        return 0

    lax.fori_loop(0, UPW // NBUF, step, 0)

    for b in range(NBUF):
        wait_stores(b)


_emb = functools.partial(
    pl.kernel,
    out_type=jax.ShapeDtypeStruct((N_FIELDS, DG, BTILES, 8, BT), jnp.float32),
    mesh=plsc.VectorSubcoreMesh(core_axis_name="c", subcore_axis_name="s"),
    scratch_types=[
        pltpu.VMEM((UPW, BT), jnp.int32),
        pltpu.VMEM((NBUF, BT, DIM), jnp.float32),
        pltpu.VMEM((NBUF, BT, DIM + 1), jnp.float32),
        pltpu.VMEM((NBUF, DG, 8, BT), jnp.float32),
        pltpu.SemaphoreType.DMA,
        pltpu.SemaphoreType.DMA((NBUF,)),
    ],
    compiler_params=pltpu.CompilerParams(
        use_tc_tiling_on_sc=False,
        needs_layout_passes=False,
        skip_device_barrier=True,
    ),
)(_body)


def kernel(indices, embedding_table):
    idx2d = indices.astype(jnp.int32).T.reshape(UNITS, BT)
    out = _emb(idx2d, embedding_table)
    return out.transpose(2, 4, 0, 1, 3).reshape(BATCH, N_FIELDS, DIM)
